# baseline (speedup 1.0000x reference)
import functools

import jax
import jax.numpy as jnp
from jax import lax
from jax.experimental import pallas as pl
from jax.experimental.pallas import tpu as pltpu
from jax.experimental.pallas import tpu_sc as plsc

SUMMARY = 50
NC, NS = 2, 16
NW = NC * NS


def _make_gather(total_rows, vocab, d, batch):
    rows_per_w = total_rows // NW
    chunk = 536
    n_chunks = rows_per_w // chunk
    assert chunk * n_chunks == rows_per_w and chunk % 8 == 0
    tb = batch // NW

    mesh = plsc.VectorSubcoreMesh(core_axis_name="c", subcore_axis_name="s")

    @functools.partial(
        pl.kernel,
        out_type=jax.ShapeDtypeStruct((total_rows, d), jnp.float32),
        mesh=mesh,
        scratch_types=[
            pltpu.VMEM((chunk,), jnp.int32),
            pltpu.VMEM((chunk, d), jnp.float32),
            pltpu.VMEM((tb,), jnp.int32),
            pltpu.VMEM((tb, d), jnp.float32),
            pltpu.VMEM((d,), jnp.float32),
            pltpu.SemaphoreType.DMA,
        ],
        compiler_params=pltpu.CompilerParams(use_tc_tiling_on_sc=False),
    )
    def gather_kernel(idx_hbm, tidx_hbm, table_hbm, tv_hbm, out_hbm,
                      idx_v, rows_v, tidx_v, tbuf_v, tv_v, sem):
        wid = lax.axis_index("s") * NC + lax.axis_index("c")
        base = wid * rows_per_w
        for c in range(n_chunks):
            off = base + c * chunk
            pltpu.sync_copy(idx_hbm.at[pl.ds(off, chunk)], idx_v)
            pltpu.async_copy(table_hbm.at[idx_v], rows_v, sem).wait()
            pltpu.sync_copy(rows_v, out_hbm.at[pl.ds(off, chunk)])

        pltpu.sync_copy(tv_hbm, tv_v)
        for k in range(d // 16):
            seg = tv_v[pl.ds(k * 16, 16)]
            for b in range(tb):
                tbuf_v[b, pl.ds(k * 16, 16)] = seg
        pltpu.sync_copy(tidx_hbm.at[pl.ds(wid * tb, tb)], tidx_v)
        pltpu.async_copy(tbuf_v, out_hbm.at[tidx_v], sem).wait()

    return gather_kernel


def kernel(x, table, type_embedding):
    b, seq = x.shape
    vocab, d = table.shape
    s = SUMMARY
    idx_full = jnp.concatenate(
        [x[:, : seq - s],
         jnp.zeros((b, 1), jnp.int32),
         x[:, seq - s:]], axis=1).reshape(-1)
    tidx = jnp.arange(b, dtype=jnp.int32) * (seq + 1) + (seq - s)
    tv = type_embedding.reshape(d)
    gather_kernel = _make_gather(b * (seq + 1), vocab, d, b)
    out_flat = gather_kernel(idx_full, tidx, table, tv)
    return out_flat.reshape(b, seq + 1, d)


# speedup vs baseline: 1.0004x; 1.0004x over previous
import functools

import jax
import jax.numpy as jnp
from jax import lax
from jax.experimental import pallas as pl
from jax.experimental.pallas import tpu as pltpu
from jax.experimental.pallas import tpu_sc as plsc

SUMMARY = 50
NC, NS = 2, 16
NW = NC * NS


def _make_gather(total_rows, vocab, d, batch):
    rows_per_w = total_rows // NW
    chunk = 536
    n_chunks = rows_per_w // chunk
    assert chunk * n_chunks == rows_per_w and chunk % 8 == 0
    tb = batch // NW

    mesh = plsc.VectorSubcoreMesh(core_axis_name="c", subcore_axis_name="s")

    @functools.partial(
        pl.kernel,
        out_type=jax.ShapeDtypeStruct((total_rows, d), jnp.float32),
        mesh=mesh,
        scratch_types=[
            pltpu.VMEM((2, chunk), jnp.int32),
            pltpu.VMEM((2, chunk, d), jnp.float32),
            pltpu.VMEM((tb,), jnp.int32),
            pltpu.VMEM((tb, d), jnp.float32),
            pltpu.VMEM((d,), jnp.float32),
            pltpu.SemaphoreType.DMA,
            pltpu.SemaphoreType.DMA,
            pltpu.SemaphoreType.DMA,
            pltpu.SemaphoreType.DMA,
            pltpu.SemaphoreType.DMA,
            pltpu.SemaphoreType.DMA,
            pltpu.SemaphoreType.DMA,
        ],
        compiler_params=pltpu.CompilerParams(use_tc_tiling_on_sc=False),
    )
    def gather_kernel(idx_hbm, tidx_hbm, table_hbm, tv_hbm, out_hbm,
                      idx_v, rows_v, tidx_v, tbuf_v, tv_v,
                      si0, si1, sg0, sg1, ss0, ss1, st):
        wid = lax.axis_index("s") * NC + lax.axis_index("c")
        base = wid * rows_per_w
        sem_i, sem_g, sem_s = (si0, si1), (sg0, sg1), (ss0, ss1)

        def start_idx(c):
            off = base + c * chunk
            return pltpu.async_copy(
                idx_hbm.at[pl.ds(off, chunk)], idx_v.at[c % 2], sem_i[c % 2])

        def start_gather(c):
            return pltpu.async_copy(
                table_hbm.at[idx_v.at[c % 2]], rows_v.at[c % 2], sem_g[c % 2])

        def start_store(c):
            off = base + c * chunk
            return pltpu.async_copy(
                rows_v.at[c % 2], out_hbm.at[pl.ds(off, chunk)], sem_s[c % 2])

        idx_d = {0: start_idx(0), 1: start_idx(1)}
        tidx_d = pltpu.async_copy(
            tidx_hbm.at[pl.ds(wid * tb, tb)], tidx_v, st)
        pltpu.sync_copy(tv_hbm, tv_v)
        for k in range(d // 16):
            seg = tv_v[pl.ds(k * 16, 16)]
            for b in range(tb):
                tbuf_v[b, pl.ds(k * 16, 16)] = seg
        tidx_d.wait()

        gat_d, sto_d = {}, {}
        for c in range(n_chunks):
            idx_d[c].wait()
            if c >= 2:
                sto_d[c - 2].wait()
            gat_d[c] = start_gather(c)
            gat_d[c].wait()
            sto_d[c] = start_store(c)
            if c + 2 < n_chunks:
                idx_d[c + 2] = start_idx(c + 2)
        sto_d[n_chunks - 2].wait()
        sto_d[n_chunks - 1].wait()

        pltpu.async_copy(tbuf_v, out_hbm.at[tidx_v], st).wait()

    return gather_kernel


def kernel(x, table, type_embedding):
    b, seq = x.shape
    vocab, d = table.shape
    s = SUMMARY
    idx_full = jnp.concatenate(
        [x[:, : seq - s],
         jnp.zeros((b, 1), jnp.int32),
         x[:, seq - s:]], axis=1).reshape(-1)
    tidx = jnp.arange(b, dtype=jnp.int32) * (seq + 1) + (seq - s)
    tv = type_embedding.reshape(d)
    gather_kernel = _make_gather(b * (seq + 1), vocab, d, b)
    out_flat = gather_kernel(idx_full, tidx, table, tv)
    return out_flat.reshape(b, seq + 1, d)
